# baseline (speedup 1.0000x reference)
import functools

import jax
import jax.numpy as jnp
from jax import lax
from jax.experimental import pallas as pl
from jax.experimental.pallas import tpu as pltpu
from jax.experimental.pallas import tpu_sc as plsc

_K = 128
_L = 16
_TR = 17
_NS = 4


def _build(batch, kdim):
    info = plsc.get_sparse_core_info()
    nc, ns = info.num_cores, info.num_subcores
    nw = nc * ns
    b_per_w = batch // nw
    chunk = 64
    n_chunks = b_per_w // chunk
    n_groups = chunk // _L
    n_j = kdim // _L

    mesh = plsc.VectorSubcoreMesh(core_axis_name="c", subcore_axis_name="s")

    @functools.partial(
        pl.kernel,
        mesh=mesh,
        out_type=jax.ShapeDtypeStruct((batch,), jnp.float32),
        compiler_params=pltpu.CompilerParams(needs_layout_passes=False,
                                             skip_device_barrier=True),
        scratch_types=[
            pltpu.VMEM((_NS, chunk), jnp.int32),
            pltpu.VMEM((_NS, chunk), jnp.int32),
            pltpu.VMEM((_NS, chunk), jnp.int32),
            pltpu.VMEM((_NS, chunk, kdim), jnp.float32),
            pltpu.VMEM((_NS, chunk, kdim), jnp.float32),
            pltpu.VMEM((_NS, chunk, kdim), jnp.float32),
            pltpu.VMEM((_NS, chunk), jnp.float32),
            pltpu.VMEM(((_L - 1) * _TR + _L,), jnp.float32),
            [pltpu.SemaphoreType.DMA] * _NS,
            [pltpu.SemaphoreType.DMA] * _NS,
            [pltpu.SemaphoreType.DMA] * _NS,
        ],
    )
    def sc_kernel(s_hbm, r_hbm, w_hbm, st_hbm, rt_hbm, wt_hbm, out_hbm,
                  idx_s, idx_r, idx_w, rows_s, rows_r, rows_w,
                  out_buf, tr_buf, semr, semi, semo):
        wid = lax.axis_index("s") * nc + lax.axis_index("c")
        base_w = wid * b_per_w

        def idx_load(ci, slot):
            base = base_w + ci * chunk
            return (
                pltpu.async_copy(s_hbm.at[pl.ds(base, chunk)],
                                 idx_s.at[slot], semi[slot]),
                pltpu.async_copy(r_hbm.at[pl.ds(base, chunk)],
                                 idx_r.at[slot], semi[slot]),
                pltpu.async_copy(w_hbm.at[pl.ds(base, chunk)],
                                 idx_w.at[slot], semi[slot]),
            )

        def fire_rows(slot):
            return (
                pltpu.async_copy(st_hbm.at[idx_s.at[slot]], rows_s.at[slot],
                                 semr[slot]),
                pltpu.async_copy(rt_hbm.at[idx_r.at[slot]], rows_r.at[slot],
                                 semr[slot]),
                pltpu.async_copy(wt_hbm.at[idx_w.at[slot]], rows_w.at[slot],
                                 semr[slot]),
            )

        def compute(slot):
            rs, rr, rw = rows_s.at[slot], rows_r.at[slot], rows_w.at[slot]
            ob = out_buf.at[slot]

            def group_body(g, carry):
                lane_ids = lax.iota(jnp.int32, _L)
                tr_idx = lane_ids * _TR
                for e0 in range(0, _L, 4):
                    accs = []
                    for e in range(e0, e0 + 4):
                        i = g * _L + e
                        parts = []
                        for j in range(n_j):
                            sv = rs[i, pl.ds(j * _L, _L)]
                            rv = rr[i, pl.ds(j * _L, _L)]
                            wv = rw[i, pl.ds(j * _L, _L)]
                            parts.append((sv + rv) * wv)
                        while len(parts) > 1:
                            parts = [parts[k] + parts[k + 1]
                                     for k in range(0, len(parts), 2)]
                        accs.append(parts[0])
                    for e in range(e0, e0 + 4):
                        plsc.store_scatter(tr_buf, [tr_idx + e], accs[e - e0])
                sums = [tr_buf[pl.ds(l * _TR, _L)] for l in range(_L)]
                while len(sums) > 1:
                    sums = [sums[k] + sums[k + 1]
                            for k in range(0, len(sums), 2)]
                prob = 1.0 / (1.0 + jnp.exp(-sums[0]))
                ob[pl.ds(g * _L, _L)] = prob
                return carry

            lax.fori_loop(0, n_groups, group_body, 0)

        pend_idx = {}
        pend_rows = {}
        pend_out = {}
        for c in range(min(3, n_chunks)):
            pend_idx[c] = idx_load(c, c % _NS)
        for c in range(min(2, n_chunks)):
            for d in pend_idx[c]:
                d.wait()
            pend_rows[c] = fire_rows(c % _NS)
        for ci in range(n_chunks):
            slot = ci % _NS
            if ci + 2 < n_chunks:
                for d in pend_idx[ci + 2]:
                    d.wait()
                pend_rows[ci + 2] = fire_rows((ci + 2) % _NS)
            if ci + 3 < n_chunks:
                pend_idx[ci + 3] = idx_load(ci + 3, (ci + 3) % _NS)
            for d in pend_rows[ci]:
                d.wait()
            if ci - 3 >= 0:
                pend_out[ci - 3].wait()
            compute(slot)
            base = base_w + ci * chunk
            pend_out[ci] = pltpu.async_copy(
                out_buf.at[slot], out_hbm.at[pl.ds(base, chunk)], semo[slot])
        for ci in range(max(0, n_chunks - 3), n_chunks):
            pend_out[ci].wait()

    return sc_kernel


@jax.jit
def kernel(s, r, w, s_table, r_table, w_table):
    batch = s.shape[0]
    fn = _build(batch, s_table.shape[1])
    s32 = s.reshape(-1).astype(jnp.int32)
    r32 = r.reshape(-1).astype(jnp.int32)
    w32 = w.reshape(-1).astype(jnp.int32)
    return fn(s32, r32, w32, s_table, r_table, w_table)
